# baseline (speedup 1.0000x reference)
import functools

import jax
import jax.numpy as jnp
from jax import lax
from jax.experimental import pallas as pl
from jax.experimental.pallas import tpu as pltpu
from jax.experimental.pallas import tpu_sc as plsc

_N_NODES = 50_000
_E = 800_000
_NLG = 800_000
_LANES = 16
_NW = 32
_BLK = 256
_NBLK_G = _E // _BLK
_NBLK_H = _NLG // _BLK
_ITERS_G = (_NBLK_G + _NW - 1) // _NW
_ITERS_H = (_NBLK_H + _NW - 1) // _NW
_PQ_WORDS = _E // _LANES
_OBW = _BLK * 64


def _vmesh():
    return plsc.VectorSubcoreMesh(core_axis_name="c", subcore_axis_name="s")


def _sc_pack_q(z32, sg, dg):
    @functools.partial(
        pl.kernel,
        mesh=_vmesh(),
        compiler_params=pltpu.CompilerParams(needs_layout_passes=False),
        out_type=jax.ShapeDtypeStruct((_PQ_WORDS,), jnp.int32),
        scratch_types=[
            pltpu.VMEM((_N_NODES,), jnp.int32),
            pltpu.VMEM((_BLK,), jnp.int32),
            pltpu.VMEM((_BLK,), jnp.int32),
            pltpu.VMEM((_LANES,), jnp.int32),
        ],
    )
    def k(z_hbm, sg_hbm, dg_hbm, pq_hbm, zv, sbuf, dbuf, obuf):
        wid = lax.axis_index("s") * 2 + lax.axis_index("c")
        pltpu.sync_copy(z_hbm, zv)

        def body(i, carry):
            b = wid + _NW * i

            @pl.when(b < _NBLK_G)
            def _():
                off = pl.multiple_of(b * _BLK, _BLK)
                pltpu.sync_copy(sg_hbm.at[pl.ds(off, _BLK)], sbuf)
                pltpu.sync_copy(dg_hbm.at[pl.ds(off, _BLK)], dbuf)
                acc = jnp.zeros((_LANES,), jnp.int32)
                for t in range(16):
                    si = sbuf[pl.ds(t * _LANES, _LANES)]
                    di = dbuf[pl.ds(t * _LANES, _LANES)]
                    zs = plsc.load_gather(zv, [si])
                    zd = plsc.load_gather(zv, [di])
                    q = (zs << 1) | zd
                    acc = acc | (q << (2 * t))
                obuf[...] = acc
                woff = pl.multiple_of(b * _LANES, _LANES)
                pltpu.sync_copy(obuf, pq_hbm.at[pl.ds(woff, _LANES)])

            return carry

        lax.fori_loop(0, _ITERS_G, body, 0)

    return k(z32, sg, dg)


def _sc_expand(pq, sh, dh, e1f, e2f, e3f):
    @functools.partial(
        pl.kernel,
        mesh=_vmesh(),
        compiler_params=pltpu.CompilerParams(needs_layout_passes=False),
        out_type=jax.ShapeDtypeStruct((_NLG * 64,), jnp.float32),
        scratch_types=[
            pltpu.VMEM((_PQ_WORDS,), jnp.int32),
            pltpu.VMEM((2, _BLK), jnp.int32),
            pltpu.VMEM((2, _BLK), jnp.int32),
            pltpu.VMEM((_BLK,), jnp.int32),
            pltpu.VMEM((128,), jnp.float32),
            pltpu.VMEM((128,), jnp.float32),
            pltpu.VMEM((128,), jnp.float32),
            pltpu.VMEM((512,), jnp.float32),
            pltpu.VMEM((2 * _OBW,), jnp.float32),
            pltpu.SemaphoreType.DMA((2,)),
            pltpu.SemaphoreType.DMA((2,)),
        ],
    )
    def k(pq_hbm, sh_hbm, dh_hbm, e1_hbm, e2_hbm, e3_hbm, out_hbm,
          pqv, shb, dhb, cbuf, e1v, e2v, e3v, tv, ob, isem, osem):
        wid = lax.axis_index("s") * 2 + lax.axis_index("c")
        pltpu.sync_copy(pq_hbm, pqv)
        pltpu.sync_copy(e1_hbm, e1v)
        pltpu.sync_copy(e2_hbm, e2v)
        pltpu.sync_copy(e3_hbm, e3v)
        for kk in range(8):
            for g in range(4):
                s = g * 16
                tv[pl.ds(kk * 64 + s, 16)] = (
                    e1v[pl.ds((kk >> 2) * 64 + s, 16)]
                    + e2v[pl.ds(((kk >> 1) & 1) * 64 + s, 16)]
                    + e3v[pl.ds((kk & 1) * 64 + s, 16)])

        def start_in(slot, b):
            off = pl.multiple_of(b * _BLK, _BLK)
            pltpu.async_copy(sh_hbm.at[pl.ds(off, _BLK)], shb.at[slot],
                             isem.at[slot])
            pltpu.async_copy(dh_hbm.at[pl.ds(off, _BLK)], dhb.at[slot],
                             isem.at[slot])

        def wait_in(slot, b):
            off = pl.multiple_of(b * _BLK, _BLK)
            pltpu.make_async_copy(sh_hbm.at[pl.ds(off, _BLK)], shb.at[slot],
                                  isem.at[slot]).wait()
            pltpu.make_async_copy(dh_hbm.at[pl.ds(off, _BLK)], dhb.at[slot],
                                  isem.at[slot]).wait()

        def start_out(slot, b):
            off = pl.multiple_of(b * _OBW, _OBW)
            pltpu.async_copy(ob.at[pl.ds(slot * _OBW, _OBW)],
                             out_hbm.at[pl.ds(off, _OBW)], osem.at[slot])

        def wait_out(slot, b):
            off = pl.multiple_of(b * _OBW, _OBW)
            pltpu.make_async_copy(ob.at[pl.ds(slot * _OBW, _OBW)],
                                  out_hbm.at[pl.ds(off, _OBW)],
                                  osem.at[slot]).wait()

        def unpack(idx):
            w = ((idx >> 8) << 4) | (idx & 15)
            p = plsc.load_gather(pqv, [w])
            return (p >> ((idx >> 3) & 30)) & 3

        start_in(0, wid)

        def body(i, carry):
            par = i & 1
            b = wid + _NW * i
            valid = b < _NBLK_H
            bn = b + _NW

            @pl.when(valid)
            def _():
                wait_in(par, b)

            @pl.when(bn < _NBLK_H)
            def _():
                start_in(1 - par, bn)

            @pl.when(valid)
            def _():
                for t in range(16):
                    a = shb[par, pl.ds(t * _LANES, _LANES)]
                    c = dhb[par, pl.ds(t * _LANES, _LANES)]
                    qa = unpack(a)
                    qc = unpack(c)
                    za = (qa >> 1) & 1
                    zb = qa & 1
                    zc = qc & 1
                    code = (((1 - (za ^ zc)) << 2)
                            | ((1 - (za ^ zb)) << 1)
                            | (1 - (zb ^ zc)))
                    cbuf[pl.ds(t * _LANES, _LANES)] = code

            @pl.when(jnp.logical_and(valid, i >= 2))
            def _():
                wait_out(par, b - 2 * _NW)

            @pl.when(valid)
            def _():
                obase = par * _OBW

                def jbody(jj, carry2):
                    j0 = jj * _LANES
                    vcode = cbuf[pl.ds(j0, _LANES)]
                    for u in range(_LANES):
                        r = vcode[u] * 64
                        d = obase + (j0 + u) * 64
                        for g in range(4):
                            ob[pl.ds(d + g * 16, 16)] = tv[pl.ds(r + g * 16, 16)]
                    return carry2

                lax.fori_loop(0, _BLK // _LANES, jbody, 0)
                start_out(par, b)

            return carry

        lax.fori_loop(0, _ITERS_H, body, 0)

        nv = (_NBLK_H - wid + _NW - 1) // _NW
        last = wid + _NW * (nv - 1)
        wait_out((nv - 1) & 1, last)
        wait_out((nv - 2) & 1, last - _NW)

    return k(pq, sh, dh, e1f, e2f, e3f)


def kernel(z, edge_index_g, edge_index_h, e1, e2, e3):
    z32 = z.astype(jnp.int32)
    sg = edge_index_g[0].astype(jnp.int32)
    dg = edge_index_g[1].astype(jnp.int32)
    sh = edge_index_h[0].astype(jnp.int32)
    dh = edge_index_h[1].astype(jnp.int32)
    pq = _sc_pack_q(z32, sg, dg)
    flat = _sc_expand(pq, sh, dh, e1.reshape(128), e2.reshape(128),
                      e3.reshape(128))
    return flat.reshape(_NLG, 64)
